# baseline (speedup 1.0000x reference)
import functools

import jax
import jax.numpy as jnp
from jax import lax
from jax.experimental import pallas as pl
from jax.experimental.pallas import tpu as pltpu
from jax.experimental.pallas import tpu_sc as plsc

NEG = -1e30
LPAD = 256
TCHUNK = 200
RESCALE = 4
QQ = 8
SC_T = 8


def _sc_gather_call(nnet, labels_pad):
    B, T, C = nnet.shape
    Lp = labels_pad.shape[1]
    info = plsc.get_sparse_core_info()
    nw = info.num_cores * info.num_subcores
    n_chunks = T // SC_T
    per_w = (n_chunks + nw - 1) // nw
    mesh = plsc.VectorSubcoreMesh(core_axis_name="c", subcore_axis_name="s")

    @functools.partial(
        pl.kernel,
        out_type=jax.ShapeDtypeStruct((T, B, LPAD), jnp.float32),
        mesh=mesh,
        scratch_types=[
            pltpu.VMEM((B, Lp), jnp.int32),
            pltpu.VMEM((B, SC_T, C), jnp.float32),
            pltpu.VMEM((B, SC_T, C), jnp.float32),
            pltpu.VMEM((SC_T, B, LPAD), jnp.float32),
            pltpu.VMEM((SC_T, B, LPAD), jnp.float32),
            pltpu.SemaphoreType.DMA,
            pltpu.SemaphoreType.DMA,
            pltpu.SemaphoreType.DMA,
            pltpu.SemaphoreType.DMA,
        ],
        compiler_params=pltpu.CompilerParams(needs_layout_passes=False),
    )
    def gather_kernel(nnet_hbm, lab_hbm, out_hbm, lab_v,
                      x0, x1, o0, o1, si0, si1, so0, so1):
        wid = lax.axis_index("s") * info.num_cores + lax.axis_index("c")
        xb, ob, sib, sob = (x0, x1), (o0, o1), (si0, si1), (so0, so1)
        pltpu.sync_copy(lab_hbm, lab_v)
        for r in range(SC_T):
            for b in range(B):
                for j in range(Lp // 16, LPAD // 16):
                    ob[0][r, b, pl.ds(j * 16, 16)] = jnp.zeros((16,), jnp.float32)
                    ob[1][r, b, pl.ds(j * 16, 16)] = jnp.zeros((16,), jnp.float32)

        def chunk_of(i):
            return jnp.minimum(wid * per_w + i, n_chunks - 1)

        def start_in(i):
            t0 = chunk_of(i) * SC_T
            return pltpu.async_copy(
                nnet_hbm.at[:, pl.ds(t0, SC_T)], xb[i % 2], sib[i % 2])

        def do_gather(i):
            x_v, o_v = xb[i % 2], ob[i % 2]

            def per_b(b, carry):
                bb = jnp.full((16,), b, jnp.int32)
                for j in range(Lp // 16):
                    idx = lab_v[b, pl.ds(j * 16, 16)]
                    for r in range(SC_T):
                        rr = jnp.full((16,), r, jnp.int32)
                        o_v[r, b, pl.ds(j * 16, 16)] = plsc.load_gather(
                            x_v, [bb, rr, idx])
                return carry

            lax.fori_loop(0, B, per_b, 0, unroll=False)

        def start_out(i):
            t0 = chunk_of(i) * SC_T
            return pltpu.async_copy(
                ob[i % 2], out_hbm.at[pl.ds(t0, SC_T)], sob[i % 2])

        hin = {0: start_in(0)}
        hout = {}
        for i in range(per_w):
            if i + 1 < per_w:
                hin[i + 1] = start_in(i + 1)
            hin[i].wait()
            if i >= 2:
                hout[i - 2].wait()
            do_gather(i)
            hout[i] = start_out(i)
        for i in (per_w - 2, per_w - 1):
            if i >= 0 and i in hout:
                hout[i].wait()

    return gather_kernel(nnet, labels_pad)


def _den_body(xf_ref, xb_ref, P_ref, PT_ref, den_ref,
              sf_ref, sb_ref, u_ref, w_ref, mf_ref, mb_ref):
    k = pl.program_id(0)
    nkh = pl.num_programs(0)
    Bb, Tc, Cc = xf_ref.shape

    @pl.when(k == 0)
    def _init():
        u_ref[...] = jnp.ones((Bb, Cc), jnp.float32)
        w_ref[...] = jnp.ones((Bb, Cc), jnp.float32)
        mf_ref[...] = jnp.full((Bb, 128), -jnp.log(float(Cc)), jnp.float32)
        mb_ref[...] = jnp.zeros((Bb, 128), jnp.float32)

    expP = jnp.exp(P_ref[...])
    expPT = jnp.exp(PT_ref[...])

    def pre_s(q, carry):
        msf, msb = carry
        xfq = xf_ref[:, pl.ds(q * QQ, QQ), :]
        mxf = jnp.max(xfq, axis=2, keepdims=True)
        sf_ref[:, pl.ds(q * QQ, QQ), :] = jnp.exp(xfq - mxf)
        xbq = xb_ref[:, pl.ds(q * QQ, QQ), :]
        mxb = jnp.max(xbq, axis=2, keepdims=True)
        sb_ref[:, pl.ds(q * QQ, QQ), :] = jnp.exp(xbq - mxb)
        return (msf + jnp.sum(mxf[:, :, 0], axis=1),
                msb + jnp.sum(mxb[:, :, 0], axis=1))

    msf, msb = lax.fori_loop(0, Tc // QQ, pre_s,
                             (jnp.zeros((Bb,), jnp.float32),
                              jnp.zeros((Bb,), jnp.float32)), unroll=False)

    def den_block(i, carry):
        u, w, mf, mb = carry
        dn = (((1,), (0,)), ((), ()))
        for j in range(RESCALE):
            t = i * RESCALE + j
            s_ft = sf_ref[:, t, :]
            u = lax.dot_general(u, expP, dn,
                                preferred_element_type=jnp.float32) * s_ft
            tb = Tc - 1 - t
            s_bt = sb_ref[:, tb, :]
            w = lax.dot_general(w * s_bt, expPT, dn,
                                preferred_element_type=jnp.float32)
        ru = jnp.max(u, axis=1, keepdims=True)
        u = u * (1.0 / ru)
        mf = mf + jnp.log(ru)[:, 0]
        rw = jnp.max(w, axis=1, keepdims=True)
        w = w * (1.0 / rw)
        mb = mb + jnp.log(rw)[:, 0]
        return u, w, mf, mb

    u, w, mf, mb = lax.fori_loop(
        0, Tc // RESCALE, den_block,
        (u_ref[...], w_ref[...], mf_ref[:, 0] + msf, mb_ref[:, 0] + msb),
        unroll=False)
    u_ref[...] = u
    w_ref[...] = w
    mf_ref[...] = jnp.broadcast_to(mf[:, None], (Bb, 128))
    mb_ref[...] = jnp.broadcast_to(mb[:, None], (Bb, 128))

    @pl.when(k == nkh - 1)
    def _den_join():
        dsum = jnp.log(jnp.sum(u * w, axis=1)) + mf + mb
        den_ref[...] = jnp.broadcast_to(dsum[:, None], (Bb, 128))


def _num_body(L, nf_ref, e_ref, den_ref,
              score_ref, tf_ref, af_ref,
              g0_ref, g1_ref, g2_ref, g3_ref, o_ref):
    k = pl.program_id(0)
    nk = pl.num_programs(0)
    Tc, Bb, _ = e_ref.shape

    @pl.when(k == 0)
    def _init():
        o_ref[...] = jnp.zeros((Bb, LPAD), jnp.float32)

    def pre_g(q, carry):
        gq = jnp.exp(e_ref[pl.ds(q * QQ, QQ)])
        z = jnp.zeros((QQ, Bb, 1), jnp.float32)
        g0_ref[pl.ds(q * QQ, QQ)] = gq
        g1_ref[pl.ds(q * QQ, QQ)] = jnp.concatenate(
            [z, gq[:, :, :-1]], axis=2)
        g2_ref[pl.ds(q * QQ, QQ)] = jnp.concatenate(
            [z, z, gq[:, :, :-2]], axis=2)
        g3_ref[pl.ds(q * QQ, QQ)] = jnp.concatenate(
            [z, z, z, gq[:, :, :-3]], axis=2)
        return carry

    lax.fori_loop(0, Tc // QQ, pre_g, 0, unroll=False)

    lane = lax.broadcasted_iota(jnp.int32, (Bb, LPAD), 1)
    gs_refs = (g0_ref, g1_ref, g2_ref, g3_ref)

    def num_block(i, o):
        t0 = k * Tc + i * RESCALE
        bs = []
        for d in range(RESCALE + 1):
            if d == 0:
                diff = jnp.zeros((Bb, LPAD), jnp.float32)
            else:
                o_s = jnp.concatenate(
                    [jnp.zeros((Bb, d), jnp.float32), o[:, :-d]], axis=1)
                diff = jnp.minimum(o_s - o, 58.0)
            m_d = (lane >= d) & (lane <= t0 + d)
            bs.append(jnp.where(m_d, jnp.exp(diff), 0.0))
        for j in range(RESCALE):
            t = i * RESCALE + j
            bs = [(bs[d] + bs[d + 1]) * gs_refs[d][t]
                  for d in range(RESCALE - j)]
        pos = lane <= t0 + RESCALE
        return o + jnp.where(pos, jnp.log(jnp.where(pos, bs[0], 1.0)), 0.0)

    o = lax.fori_loop(0, Tc // RESCALE, num_block, o_ref[...], unroll=False)
    o_ref[...] = o

    @pl.when(k == nk - 1)
    def _fin():
        num = o[:, L - 1:L]
        den = den_ref[:, 0:1]
        sc = num - den
        nf = nf_ref[...]
        okm = jnp.isfinite(sc) & (sc > NEG / 2)
        score_ref[0, 0] = jnp.sum(jnp.where(okm, sc, 0.0))
        tf_ref[0, 0] = jnp.sum(jnp.where(okm, nf, 0))
        af_ref[0, 0] = jnp.sum(nf)


def _den_call(nnet, P, PT):
    B, T, C = nnet.shape
    nk = T // TCHUNK
    return pl.pallas_call(
        _den_body,
        grid=(nk // 2,),
        in_specs=[
            pl.BlockSpec((B, TCHUNK, C), lambda k: (0, k, 0)),
            pl.BlockSpec((B, TCHUNK, C), lambda k: (0, nk - 1 - k, 0)),
            pl.BlockSpec((C, C), lambda k: (0, 0)),
            pl.BlockSpec((C, C), lambda k: (0, 0)),
        ],
        out_specs=pl.BlockSpec((B, 128), lambda k: (0, 0)),
        out_shape=jax.ShapeDtypeStruct((B, 128), jnp.float32),
        scratch_shapes=[
            pltpu.VMEM((B, TCHUNK, C), jnp.float32),
            pltpu.VMEM((B, TCHUNK, C), jnp.float32),
            pltpu.VMEM((B, C), jnp.float32),
            pltpu.VMEM((B, C), jnp.float32),
            pltpu.VMEM((B, 128), jnp.float32),
            pltpu.VMEM((B, 128), jnp.float32),
        ],
    )(nnet, nnet, P, PT)


def _num_call(L, nf2, e_t, den):
    T, B, _ = e_t.shape
    nk = T // TCHUNK
    return pl.pallas_call(
        functools.partial(_num_body, L),
        grid=(nk,),
        in_specs=[
            pl.BlockSpec((B, 1), lambda k: (0, 0)),
            pl.BlockSpec((TCHUNK, B, LPAD), lambda k: (k, 0, 0)),
            pl.BlockSpec((B, 128), lambda k: (0, 0)),
        ],
        out_specs=[
            pl.BlockSpec(memory_space=pltpu.SMEM),
            pl.BlockSpec(memory_space=pltpu.SMEM),
            pl.BlockSpec(memory_space=pltpu.SMEM),
        ],
        out_shape=[
            jax.ShapeDtypeStruct((1, 1), jnp.float32),
            jax.ShapeDtypeStruct((1, 1), jnp.int32),
            jax.ShapeDtypeStruct((1, 1), jnp.int32),
        ],
        scratch_shapes=[
            pltpu.VMEM((TCHUNK, B, LPAD), jnp.float32),
            pltpu.VMEM((TCHUNK, B, LPAD), jnp.float32),
            pltpu.VMEM((TCHUNK, B, LPAD), jnp.float32),
            pltpu.VMEM((TCHUNK, B, LPAD), jnp.float32),
            pltpu.VMEM((B, LPAD), jnp.float32),
        ],
    )(nf2, e_t, den)


def kernel(nnet_output, labels, supervision_segments, P):
    B, T, C = nnet_output.shape
    L = labels.shape[1]
    lp = (-L) % 16
    labels_pad = jnp.pad(labels, ((0, 0), (0, lp)))
    e_t = _sc_gather_call(nnet_output, labels_pad)
    den = _den_call(nnet_output, P, P.T)
    nf2 = supervision_segments[:, 2:3]
    score, tf, af = _num_call(L, nf2, e_t, den)
    return score[0, 0], tf[0, 0], af[0, 0]


# speedup vs baseline: 1.0014x; 1.0014x over previous
import functools

import jax
import jax.numpy as jnp
from jax import lax
from jax.experimental import pallas as pl
from jax.experimental.pallas import tpu as pltpu
from jax.experimental.pallas import tpu_sc as plsc

NEG = -1e30
LPAD = 256
TCHUNK = 200
RESCALE = 4
QQ = 8
SC_T = 8


def _sc_gather_call(nnet, labels_pad):
    B, T, C = nnet.shape
    Lp = labels_pad.shape[1]
    info = plsc.get_sparse_core_info()
    nw = info.num_cores * info.num_subcores
    n_chunks = T // SC_T
    per_w = (n_chunks + nw - 1) // nw
    mesh = plsc.VectorSubcoreMesh(core_axis_name="c", subcore_axis_name="s")

    @functools.partial(
        pl.kernel,
        out_type=jax.ShapeDtypeStruct((T, B, LPAD), jnp.float32),
        mesh=mesh,
        scratch_types=[
            pltpu.VMEM((B, Lp), jnp.int32),
            pltpu.VMEM((B, SC_T, C), jnp.float32),
            pltpu.VMEM((B, SC_T, C), jnp.float32),
            pltpu.VMEM((SC_T, B, LPAD), jnp.float32),
            pltpu.VMEM((SC_T, B, LPAD), jnp.float32),
            pltpu.SemaphoreType.DMA,
            pltpu.SemaphoreType.DMA,
            pltpu.SemaphoreType.DMA,
            pltpu.SemaphoreType.DMA,
        ],
        compiler_params=pltpu.CompilerParams(needs_layout_passes=False),
    )
    def gather_kernel(nnet_hbm, lab_hbm, out_hbm, lab_v,
                      x0, x1, o0, o1, si0, si1, so0, so1):
        wid = lax.axis_index("s") * info.num_cores + lax.axis_index("c")
        xb, ob, sib, sob = (x0, x1), (o0, o1), (si0, si1), (so0, so1)
        pltpu.sync_copy(lab_hbm, lab_v)
        for r in range(SC_T):
            for b in range(B):
                for j in range(Lp // 16, LPAD // 16):
                    ob[0][r, b, pl.ds(j * 16, 16)] = jnp.zeros((16,), jnp.float32)
                    ob[1][r, b, pl.ds(j * 16, 16)] = jnp.zeros((16,), jnp.float32)

        def chunk_of(i):
            return jnp.minimum(wid * per_w + i, n_chunks - 1)

        def start_in(i):
            t0 = chunk_of(i) * SC_T
            return pltpu.async_copy(
                nnet_hbm.at[:, pl.ds(t0, SC_T)], xb[i % 2], sib[i % 2])

        def do_gather(i):
            x_v, o_v = xb[i % 2], ob[i % 2]

            def per_b(b, carry):
                bb = jnp.full((16,), b, jnp.int32)
                for j in range(Lp // 16):
                    idx = lab_v[b, pl.ds(j * 16, 16)]
                    for r in range(SC_T):
                        rr = jnp.full((16,), r, jnp.int32)
                        o_v[r, b, pl.ds(j * 16, 16)] = plsc.load_gather(
                            x_v, [bb, rr, idx])
                return carry

            lax.fori_loop(0, B, per_b, 0, unroll=False)

        def start_out(i):
            t0 = chunk_of(i) * SC_T
            return pltpu.async_copy(
                ob[i % 2], out_hbm.at[pl.ds(t0, SC_T)], sob[i % 2])

        hin = {0: start_in(0)}
        hout = {}
        for i in range(per_w):
            if i + 1 < per_w:
                hin[i + 1] = start_in(i + 1)
            hin[i].wait()
            if i >= 2:
                hout[i - 2].wait()
            do_gather(i)
            hout[i] = start_out(i)
        for i in (per_w - 2, per_w - 1):
            if i >= 0 and i in hout:
                hout[i].wait()

    return gather_kernel(nnet, labels_pad)


def _den_body(xf_ref, xb_ref, P_ref, PT_ref, den_ref,
              sf_ref, sb_ref, u_ref, w_ref, mf_ref, mb_ref):
    k = pl.program_id(0)
    nkh = pl.num_programs(0)
    Bb, Tc, Cc = xf_ref.shape

    @pl.when(k == 0)
    def _init():
        u_ref[...] = jnp.ones((Bb, Cc), jnp.float32)
        w_ref[...] = jnp.ones((Bb, Cc), jnp.float32)
        mf_ref[...] = jnp.full((Bb, 128), -jnp.log(float(Cc)), jnp.float32)
        mb_ref[...] = jnp.zeros((Bb, 128), jnp.float32)

    expP = jnp.exp(P_ref[...])
    expPT = jnp.exp(PT_ref[...])

    def pre_s(q, carry):
        msf, msb = carry
        xfq = xf_ref[:, pl.ds(q * QQ, QQ), :]
        mxf = jnp.max(xfq, axis=2, keepdims=True)
        sf_ref[:, pl.ds(q * QQ, QQ), :] = jnp.exp(xfq - mxf)
        xbq = xb_ref[:, pl.ds(q * QQ, QQ), :]
        mxb = jnp.max(xbq, axis=2, keepdims=True)
        sb_ref[:, pl.ds(q * QQ, QQ), :] = jnp.exp(xbq - mxb)
        return (msf + jnp.sum(mxf[:, :, 0], axis=1),
                msb + jnp.sum(mxb[:, :, 0], axis=1))

    msf, msb = lax.fori_loop(0, Tc // QQ, pre_s,
                             (jnp.zeros((Bb,), jnp.float32),
                              jnp.zeros((Bb,), jnp.float32)), unroll=False)

    def den_block(i, carry):
        u, w, mf, mb = carry
        dn = (((1,), (0,)), ((), ()))
        for j in range(RESCALE):
            t = i * RESCALE + j
            s_ft = sf_ref[:, t, :]
            u = lax.dot_general(u, expP, dn,
                                preferred_element_type=jnp.float32) * s_ft
            tb = Tc - 1 - t
            s_bt = sb_ref[:, tb, :]
            w = lax.dot_general(w * s_bt, expPT, dn,
                                preferred_element_type=jnp.float32)
        ru = jnp.max(u, axis=1, keepdims=True)
        u = u * (1.0 / ru)
        mf = mf + jnp.log(ru)[:, 0]
        rw = jnp.max(w, axis=1, keepdims=True)
        w = w * (1.0 / rw)
        mb = mb + jnp.log(rw)[:, 0]
        return u, w, mf, mb

    u, w, mf, mb = lax.fori_loop(
        0, Tc // RESCALE, den_block,
        (u_ref[...], w_ref[...], mf_ref[:, 0] + msf, mb_ref[:, 0] + msb),
        unroll=False)
    u_ref[...] = u
    w_ref[...] = w
    mf_ref[...] = jnp.broadcast_to(mf[:, None], (Bb, 128))
    mb_ref[...] = jnp.broadcast_to(mb[:, None], (Bb, 128))

    @pl.when(k == nkh - 1)
    def _den_join():
        dsum = jnp.log(jnp.sum(u * w, axis=1)) + mf + mb
        den_ref[...] = jnp.broadcast_to(dsum[:, None], (Bb, 128))


def _num_body(L, nf_ref, e_ref, den_ref,
              score_ref, tf_ref, af_ref,
              g0_ref, g1_ref, g2_ref, g3_ref, o_ref):
    k = pl.program_id(0)
    nk = pl.num_programs(0)
    Tc, Bb, _ = e_ref.shape

    @pl.when(k == 0)
    def _init():
        o_ref[...] = jnp.zeros((Bb, LPAD), jnp.float32)

    def pre_g(q, carry):
        gq = jnp.exp(e_ref[pl.ds(q * QQ, QQ)])
        z = jnp.zeros((QQ, Bb, 1), jnp.float32)
        g0_ref[pl.ds(q * QQ, QQ)] = gq
        g1_ref[pl.ds(q * QQ, QQ)] = jnp.concatenate(
            [z, gq[:, :, :-1]], axis=2)
        g2_ref[pl.ds(q * QQ, QQ)] = jnp.concatenate(
            [z, z, gq[:, :, :-2]], axis=2)
        g3_ref[pl.ds(q * QQ, QQ)] = jnp.concatenate(
            [z, z, z, gq[:, :, :-3]], axis=2)
        return carry

    lax.fori_loop(0, Tc // QQ, pre_g, 0, unroll=False)

    lane = lax.broadcasted_iota(jnp.int32, (Bb, LPAD), 1)
    gs_refs = (g0_ref, g1_ref, g2_ref, g3_ref)

    def num_block(i, o):
        t0 = k * Tc + i * RESCALE
        bs = []
        for d in range(RESCALE + 1):
            if d == 0:
                diff = jnp.zeros((Bb, LPAD), jnp.float32)
            else:
                o_s = jnp.concatenate(
                    [jnp.zeros((Bb, d), jnp.float32), o[:, :-d]], axis=1)
                diff = jnp.minimum(o_s - o, 58.0)
            m_d = (lane >= d) & (lane <= t0 + d)
            bs.append(jnp.where(m_d, jnp.exp(diff), 0.0))
        for j in range(RESCALE):
            t = i * RESCALE + j
            bs = [(bs[d] + bs[d + 1]) * gs_refs[d][t]
                  for d in range(RESCALE - j)]
        pos = lane <= t0 + RESCALE
        return o + jnp.where(pos, jnp.log(jnp.where(pos, bs[0], 1.0)), 0.0)

    o = lax.fori_loop(0, Tc // RESCALE, num_block, o_ref[...], unroll=False)
    o_ref[...] = o

    @pl.when(k == nk - 1)
    def _fin():
        num = o[:, L - 1:L]
        den = den_ref[:, 0:1]
        sc = num - den
        nf = nf_ref[...]
        okm = jnp.isfinite(sc) & (sc > NEG / 2)
        score_ref[0, 0] = jnp.sum(jnp.where(okm, sc, 0.0))
        tf_ref[0, 0] = jnp.sum(jnp.where(okm, nf, 0))
        af_ref[0, 0] = jnp.sum(nf)


def _den_call(nnet, P, PT):
    B, T, C = nnet.shape
    nk = T // TCHUNK
    return pl.pallas_call(
        _den_body,
        grid=(nk // 2,),
        in_specs=[
            pl.BlockSpec((B, TCHUNK, C), lambda k: (0, k, 0)),
            pl.BlockSpec((B, TCHUNK, C), lambda k: (0, nk - 1 - k, 0)),
            pl.BlockSpec((C, C), lambda k: (0, 0)),
            pl.BlockSpec((C, C), lambda k: (0, 0)),
        ],
        out_specs=pl.BlockSpec((B, 128), lambda k: (0, 0)),
        out_shape=jax.ShapeDtypeStruct((B, 128), jnp.float32),
        scratch_shapes=[
            pltpu.VMEM((B, TCHUNK, C), jnp.float32),
            pltpu.VMEM((B, TCHUNK, C), jnp.float32),
            pltpu.VMEM((B, C), jnp.float32),
            pltpu.VMEM((B, C), jnp.float32),
            pltpu.VMEM((B, 128), jnp.float32),
            pltpu.VMEM((B, 128), jnp.float32),
        ],
    )(nnet, nnet, P, PT)


def _num_call(L, nf2, e_t, den):
    T, B, _ = e_t.shape
    nk = T // TCHUNK
    return pl.pallas_call(
        functools.partial(_num_body, L),
        grid=(nk,),
        in_specs=[
            pl.BlockSpec((B, 1), lambda k: (0, 0)),
            pl.BlockSpec((TCHUNK, B, LPAD), lambda k: (k, 0, 0)),
            pl.BlockSpec((B, 128), lambda k: (0, 0)),
        ],
        out_specs=[
            pl.BlockSpec(memory_space=pltpu.SMEM),
            pl.BlockSpec(memory_space=pltpu.SMEM),
            pl.BlockSpec(memory_space=pltpu.SMEM),
        ],
        out_shape=[
            jax.ShapeDtypeStruct((1, 1), jnp.float32),
            jax.ShapeDtypeStruct((1, 1), jnp.int32),
            jax.ShapeDtypeStruct((1, 1), jnp.int32),
        ],
        scratch_shapes=[
            pltpu.VMEM((TCHUNK, B, LPAD), jnp.float32),
            pltpu.VMEM((TCHUNK, B, LPAD), jnp.float32),
            pltpu.VMEM((TCHUNK, B, LPAD), jnp.float32),
            pltpu.VMEM((TCHUNK, B, LPAD), jnp.float32),
            pltpu.VMEM((B, LPAD), jnp.float32),
        ],
    )(nf2, e_t, den)


def kernel(nnet_output, labels, supervision_segments, P):
    B, T, C = nnet_output.shape
    L = labels.shape[1]
    lp = (-L) % 16
    labels_pad = jnp.pad(labels, ((0, 0), (0, lp)))
    den = _den_call(nnet_output, P, P.T)
    e_t = _sc_gather_call(nnet_output, labels_pad)
    nf2 = supervision_segments[:, 2:3]
    score, tf, af = _num_call(L, nf2, e_t, den)
    return score[0, 0], tf[0, 0], af[0, 0]
